# baseline (speedup 1.0000x reference)
import jax
import jax.numpy as jnp
from jax import lax
from jax.experimental import pallas as pl
from jax.experimental.pallas import tpu as pltpu
from jax.experimental.pallas import tpu_sc as plsc

NC = 2
NS = 16
NW = NC * NS
L = 16
B = 16384
D = 64
BPW = B // NW
C = 128
NSL = D // L


def _sc_body(h_hbm, r_hbm, t_hbm, e1_hbm, e2_hbm, r1_hbm, r2_hbm, out_hbm,
             hidx_v, ridx_v, tidx_v,
             e1h_v, e2h_v, r1_v, r2_v, e1t_v, e2t_v, out_v, sem):
    cid = lax.axis_index("c")
    sid = lax.axis_index("s")
    wid = sid * NC + cid
    base = wid * BPW
    lane = lax.iota(jnp.int32, L)

    for j in range(BPW // C):
        off = base + j * C
        pltpu.sync_copy(h_hbm.at[pl.ds(off, C)], hidx_v)
        pltpu.sync_copy(r_hbm.at[pl.ds(off, C)], ridx_v)
        pltpu.sync_copy(t_hbm.at[pl.ds(off, C)], tidx_v)
        cps = [
            pltpu.async_copy(e1_hbm.at[hidx_v], e1h_v, sem),
            pltpu.async_copy(e2_hbm.at[hidx_v], e2h_v, sem),
            pltpu.async_copy(r1_hbm.at[ridx_v], r1_v, sem),
            pltpu.async_copy(r2_hbm.at[ridx_v], r2_v, sem),
            pltpu.async_copy(e1_hbm.at[tidx_v], e1t_v, sem),
            pltpu.async_copy(e2_hbm.at[tidx_v], e2t_v, sem),
        ]
        for cp in cps:
            cp.wait()

        def group(g, carry):
            res = jnp.zeros((L,), jnp.float32)
            for k in range(L):
                i = g * L + k
                acc = jnp.zeros((L,), jnp.float32)
                for s in range(NSL):
                    sl = pl.ds(s * L, L)
                    acc = (acc
                           + e1h_v[i, sl] * r1_v[i, sl] * e2t_v[i, sl]
                           + e2h_v[i, sl] * r2_v[i, sl] * e1t_v[i, sl])
                res = jnp.where(lane == k, jnp.sum(acc), res)
            out_v[pl.ds(g * L, L)] = res * 0.5
            return carry

        lax.fori_loop(0, C // L, group, 0)
        pltpu.sync_copy(out_v, out_hbm.at[pl.ds(off, C)])


def kernel(h_idx, r_idx, t_idx, E1, E2, R1, R2):
    h = h_idx.astype(jnp.int32)
    r = r_idx.astype(jnp.int32)
    t = t_idx.astype(jnp.int32)
    mesh = plsc.VectorSubcoreMesh(core_axis_name="c", subcore_axis_name="s")
    fn = pl.kernel(
        _sc_body,
        mesh=mesh,
        compiler_params=pltpu.CompilerParams(
            needs_layout_passes=False, use_tc_tiling_on_sc=False),
        out_type=jax.ShapeDtypeStruct((B,), jnp.float32),
        scratch_types=[
            pltpu.VMEM((C,), jnp.int32),
            pltpu.VMEM((C,), jnp.int32),
            pltpu.VMEM((C,), jnp.int32),
            pltpu.VMEM((C, D), jnp.float32),
            pltpu.VMEM((C, D), jnp.float32),
            pltpu.VMEM((C, D), jnp.float32),
            pltpu.VMEM((C, D), jnp.float32),
            pltpu.VMEM((C, D), jnp.float32),
            pltpu.VMEM((C, D), jnp.float32),
            pltpu.VMEM((C,), jnp.float32),
            pltpu.SemaphoreType.DMA,
        ],
    )
    return fn(h, r, t, E1, E2, R1, R2)
